# baseline (speedup 1.0000x reference)
import functools

import jax
import jax.numpy as jnp
from jax import lax
from jax.experimental import pallas as pl
from jax.experimental.pallas import tpu as pltpu
from jax.experimental.pallas import tpu_sc as plsc

_SIGMA_DIM = 4
_EMB_DIM = 32
_ROW = _SIGMA_DIM + _EMB_DIM
_L = 16
_CH = 128
_W = 128


def _body(nc, bpw, nslice, keys_hbm, table_hbm, params_hbm, out_hbm,
          keys_v, idx_v, sl_v, out_v, params_v, sem):
  wid = lax.axis_index("s") * nc + lax.axis_index("c")
  base = wid * bpw
  half = bpw // 4

  pltpu.sync_copy(keys_hbm.at[pl.ds(base, bpw)], keys_v)
  pltpu.sync_copy(params_hbm, params_v)

  p0 = params_v[pl.ds(0, _L)]
  p1 = params_v[pl.ds(_L, _L)]
  p2 = params_v[pl.ds(2 * _L, _L)]
  vc = [p0[c] for c in range(_L)] + [p1[c] for c in range(_L)]
  sk = [p2[c] for c in range(_SIGMA_DIM)]
  bias = p2[_SIGMA_DIM]

  for h in range(4):
    hbase = h * half

    def build(g, carry):
      i16 = g * _L + lax.iota(jnp.int32, _L)
      k16 = keys_v[pl.ds(hbase + g * _L, _L)]
      s0 = (k16 * _ROW) >> 7
      s1 = jnp.minimum(s0 + 1, nslice - 1)
      p = i16 * 2
      plsc.store_scatter(idx_v, [p >> 7, p & (_CH - 1)], s0)
      plsc.store_scatter(idx_v, [(p + 1) >> 7, (p + 1) & (_CH - 1)], s1)
      return carry

    lax.fori_loop(0, half // _L, build, 0)

    nch = (half * 2) // _CH
    copies = [
        pltpu.async_copy(table_hbm.at[idx_v.at[j]],
                         sl_v.at[pl.ds(j * _CH, _CH)], sem)
        for j in range(nch)
    ]
    for c in copies:
      c.wait()

    def group(g, carry):
      i16 = g * _L + lax.iota(jnp.int32, _L)
      k16 = keys_v[pl.ds(hbase + g * _L, _L)]
      wbase = i16 * (2 * _W) + ((k16 * _ROW) & (_W - 1))
      acc = jnp.full((_L,), 0.0, jnp.float32)
      for c in range(_SIGMA_DIM):
        w = wbase + c
        acc = acc + sk[c] * plsc.load_gather(sl_v, [w >> 7, w & (_W - 1)])
      s = 1.0 / (1.0 + jnp.exp(-(acc + bias)))
      one_m_s = 1.0 - s
      for c in range(_EMB_DIM):
        w = wbase + (_SIGMA_DIM + c)
        v = plsc.load_gather(sl_v, [w >> 7, w & (_W - 1)])
        o = s * vc[c] + one_m_s * v
        plsc.store_scatter(out_v, [hbase + i16, jnp.full((_L,), c, jnp.int32)], o)
      return carry

    lax.fori_loop(0, half // _L, group, 0)

  pltpu.sync_copy(out_v, out_hbm.at[pl.ds(base, bpw)])


def kernel(keys, table, vc, sigma_kernel, sigma_bias):
  keys = keys.astype(jnp.int32)
  num_emb = table.shape[0]
  nslice = num_emb * _ROW // _W
  table128 = table.reshape(nslice, _W)
  params = jnp.concatenate(
      [vc, sigma_kernel, sigma_bias, jnp.zeros((11,), jnp.float32)])
  info = plsc.get_sparse_core_info()
  nc, ns = info.num_cores, info.num_subcores
  nw = nc * ns
  batch = keys.shape[0]
  bpw = batch // nw

  mesh = plsc.VectorSubcoreMesh(core_axis_name="c", subcore_axis_name="s")
  run = pl.kernel(
      functools.partial(_body, nc, bpw, nslice),
      out_type=jax.ShapeDtypeStruct((batch, _EMB_DIM), jnp.float32),
      mesh=mesh,
      compiler_params=pltpu.CompilerParams(needs_layout_passes=False),
      scratch_types=[
          pltpu.VMEM((bpw,), jnp.int32),
          pltpu.VMEM((bpw // _CH, _CH), jnp.int32),
          pltpu.VMEM((bpw // 2, _W), jnp.float32),
          pltpu.VMEM((bpw, _EMB_DIM), jnp.float32),
          pltpu.VMEM((48,), jnp.float32),
          pltpu.SemaphoreType.DMA,
      ],
  )
  return run(keys, table128, params)


# speedup vs baseline: 1.7596x; 1.7596x over previous
import functools

import jax
import jax.numpy as jnp
from jax import lax
from jax.experimental import pallas as pl
from jax.experimental.pallas import tpu as pltpu
from jax.experimental.pallas import tpu_sc as plsc

_SIGMA_DIM = 4
_EMB_DIM = 32
_ROW = _SIGMA_DIM + _EMB_DIM
_L = 16
_TH = 8


def _body(nc, bpw, keys_hbm, table_hbm, params_hbm, out_hbm,
          keys_v, rows_v, out_v, params_v, sem):
  wid = lax.axis_index("s") * nc + lax.axis_index("c")
  base = wid * bpw
  half = bpw // 2
  ngr = half // _L

  pltpu.sync_copy(keys_hbm.at[pl.ds(base, bpw)], keys_v)
  pltpu.sync_copy(params_hbm, params_v)

  p0 = params_v[pl.ds(0, _L)]
  p1 = params_v[pl.ds(_L, _L)]
  p2 = params_v[pl.ds(2 * _L, _L)]
  vc = [p0[c] for c in range(_L)] + [p1[c] for c in range(_L)]
  sk = [p2[c] for c in range(_SIGMA_DIM)]
  bias = p2[_SIGMA_DIM]

  def fetch(hbase, g, carry):
    k16 = keys_v[pl.ds(hbase + g * _L, _L)]
    t16 = k16 >> 3
    r16 = k16 & (_TH - 1)
    copies = []
    for j in range(_L):
      i = g * _L + j
      copies.append(pltpu.async_copy(
          table_hbm.at[pl.ds(t16[j], 1), pl.ds(r16[j], 1), :],
          rows_v.at[pl.ds(i, 1)], sem))
    for c in copies:
      c.wait()
    return carry

  def group(hbase, g, carry):
    i16 = g * _L + lax.iota(jnp.int32, _L)
    z16 = jnp.full((_L,), 0, jnp.int32)
    acc = jnp.full((_L,), 0.0, jnp.float32)
    for c in range(_SIGMA_DIM):
      col = jnp.full((_L,), c, jnp.int32)
      acc = acc + sk[c] * plsc.load_gather(rows_v, [i16, z16, col])
    s = 1.0 / (1.0 + jnp.exp(-(acc + bias)))
    one_m_s = 1.0 - s
    for c in range(_EMB_DIM):
      col = jnp.full((_L,), _SIGMA_DIM + c, jnp.int32)
      v = plsc.load_gather(rows_v, [i16, z16, col])
      o = s * vc[c] + one_m_s * v
      plsc.store_scatter(out_v, [hbase + i16, jnp.full((_L,), c, jnp.int32)], o)
    return carry

  for h in range(2):
    hbase = h * half
    lax.fori_loop(0, ngr, functools.partial(fetch, hbase), 0)
    lax.fori_loop(0, ngr, functools.partial(group, hbase), 0)

  pltpu.sync_copy(out_v, out_hbm.at[pl.ds(base, bpw)])


def kernel(keys, table, vc, sigma_kernel, sigma_bias):
  keys = keys.astype(jnp.int32)
  num_emb = table.shape[0]
  table3 = table.reshape(num_emb // _TH, _TH, _ROW)
  params = jnp.concatenate(
      [vc, sigma_kernel, sigma_bias, jnp.zeros((11,), jnp.float32)])
  info = plsc.get_sparse_core_info()
  nc, ns = info.num_cores, info.num_subcores
  nw = nc * ns
  batch = keys.shape[0]
  bpw = batch // nw

  mesh = plsc.VectorSubcoreMesh(core_axis_name="c", subcore_axis_name="s")
  run = pl.kernel(
      functools.partial(_body, nc, bpw),
      out_type=jax.ShapeDtypeStruct((batch, _EMB_DIM), jnp.float32),
      mesh=mesh,
      compiler_params=pltpu.CompilerParams(needs_layout_passes=False),
      scratch_types=[
          pltpu.VMEM((bpw,), jnp.int32),
          pltpu.VMEM((bpw // 2, 1, _ROW), jnp.float32),
          pltpu.VMEM((bpw, _EMB_DIM), jnp.float32),
          pltpu.VMEM((48,), jnp.float32),
          pltpu.SemaphoreType.DMA,
      ],
  )
  return run(keys, table3, params)


# speedup vs baseline: 1.7716x; 1.0068x over previous
import functools

import jax
import jax.numpy as jnp
from jax import lax
from jax.experimental import pallas as pl
from jax.experimental.pallas import tpu as pltpu
from jax.experimental.pallas import tpu_sc as plsc

_SIGMA_DIM = 4
_EMB_DIM = 32
_ROW = _SIGMA_DIM + _EMB_DIM
_L = 16
_TH = 8


def _body(nc, bpw, keys_hbm, table_hbm, params_hbm, out_hbm,
          keys_v, rows_v, out_v, params_v, sem):
  wid = lax.axis_index("s") * nc + lax.axis_index("c")
  base = wid * bpw
  half = bpw // 2
  ngr = half // _L

  pltpu.sync_copy(keys_hbm.at[pl.ds(base, bpw)], keys_v)
  pltpu.sync_copy(params_hbm, params_v)

  p0 = params_v[pl.ds(0, _L)]
  p1 = params_v[pl.ds(_L, _L)]
  p2 = params_v[pl.ds(2 * _L, _L)]
  vc = [p0[c] for c in range(_L)] + [p1[c] for c in range(_L)]
  sk = [p2[c] for c in range(_SIGMA_DIM)]
  bias = p2[_SIGMA_DIM]

  def fetch(hbase):
    copies = []
    for g in range(ngr):
      k16 = keys_v[pl.ds(hbase + g * _L, _L)]
      t16 = k16 >> 3
      r16 = k16 & (_TH - 1)
      for j in range(_L):
        i = g * _L + j
        copies.append(pltpu.async_copy(
            table_hbm.at[pl.ds(t16[j], 1), pl.ds(r16[j], 1), :],
            rows_v.at[pl.ds(i, 1)], sem))
    for c in copies:
      c.wait()

  def group(hbase, g, carry):
    i16 = g * _L + lax.iota(jnp.int32, _L)
    z16 = jnp.full((_L,), 0, jnp.int32)
    acc = jnp.full((_L,), 0.0, jnp.float32)
    for c in range(_SIGMA_DIM):
      col = jnp.full((_L,), c, jnp.int32)
      acc = acc + sk[c] * plsc.load_gather(rows_v, [i16, z16, col])
    s = 1.0 / (1.0 + jnp.exp(-(acc + bias)))
    one_m_s = 1.0 - s
    for c in range(_EMB_DIM):
      col = jnp.full((_L,), _SIGMA_DIM + c, jnp.int32)
      v = plsc.load_gather(rows_v, [i16, z16, col])
      o = s * vc[c] + one_m_s * v
      plsc.store_scatter(out_v, [hbase + i16, jnp.full((_L,), c, jnp.int32)], o)
    return carry

  for h in range(2):
    hbase = h * half
    fetch(hbase)
    lax.fori_loop(0, ngr, functools.partial(group, hbase), 0)

  pltpu.sync_copy(out_v, out_hbm.at[pl.ds(base, bpw)])


def kernel(keys, table, vc, sigma_kernel, sigma_bias):
  keys = keys.astype(jnp.int32)
  num_emb = table.shape[0]
  table3 = table.reshape(num_emb // _TH, _TH, _ROW)
  params = jnp.concatenate(
      [vc, sigma_kernel, sigma_bias, jnp.zeros((11,), jnp.float32)])
  info = plsc.get_sparse_core_info()
  nc, ns = info.num_cores, info.num_subcores
  nw = nc * ns
  batch = keys.shape[0]
  bpw = batch // nw

  mesh = plsc.VectorSubcoreMesh(core_axis_name="c", subcore_axis_name="s")
  run = pl.kernel(
      functools.partial(_body, nc, bpw),
      out_type=jax.ShapeDtypeStruct((batch, _EMB_DIM), jnp.float32),
      mesh=mesh,
      compiler_params=pltpu.CompilerParams(needs_layout_passes=False),
      scratch_types=[
          pltpu.VMEM((bpw,), jnp.int32),
          pltpu.VMEM((bpw // 2, 1, _ROW), jnp.float32),
          pltpu.VMEM((bpw, _EMB_DIM), jnp.float32),
          pltpu.VMEM((48,), jnp.float32),
          pltpu.SemaphoreType.DMA,
      ],
  )
  return run(keys, table3, params)
